# Initial kernel scaffold; baseline (speedup 1.0000x reference)
import jax
import jax.numpy as jnp
from jax.experimental import pallas as pl


def kernel(idx, targets, table):
    raise NotImplementedError("write your pallas kernel here")


# speedup vs baseline: 1.6066x; 1.6066x over previous
import functools

import jax
import jax.numpy as jnp
from jax import lax
from jax.experimental import pallas as pl
from jax.experimental.pallas import tpu as pltpu
from jax.experimental.pallas import tpu_sc as plsc

VOCAB_N = 1000
ROWS_N = 51200
NC = 2
NS = 16
NW = NC * NS
PER_W = ROWS_N // NW
CH = 32
NCH = PER_W // CH
LSE_PAD = 1024


def _lse_body(tpad_ref, out_ref):
    x = tpad_ref[...]
    m = jnp.max(x, axis=1)
    s = jnp.sum(jnp.exp(x - m[:, None]), axis=1)
    out_ref[...] = m + jnp.log(s)


def _mean_body(part_ref, out_ref):
    out_ref[...] = (jnp.sum(part_ref[...]) * (1.0 / ROWS_N))[None, None]


def _sc_body(idx_hbm, tgt_hbm, lse_hbm, table_hbm, out_hbm, part_hbm,
             idx_v, tgt_v, lse_v, rows_v, part_v, sem):
    wid = lax.axis_index("s") * NC + lax.axis_index("c")
    base = wid * PER_W
    pltpu.sync_copy(idx_hbm.at[pl.ds(base, PER_W)], idx_v)
    pltpu.sync_copy(tgt_hbm.at[pl.ds(base, PER_W)], tgt_v)
    pltpu.sync_copy(lse_hbm, lse_v)

    def chunk_body(c, acc):
        start = c * CH
        pltpu.async_copy(
            table_hbm.at[idx_v.at[pl.ds(start, CH)]], rows_v, sem).wait()
        pltpu.sync_copy(rows_v, out_hbm.at[pl.ds(base + start, CH)])
        for g in range(CH // 16):
            rid = lax.iota(jnp.int32, 16) + g * 16
            cid = tgt_v[pl.ds(start + g * 16, 16)]
            tl = plsc.load_gather(rows_v, [rid, cid])
            ii = idx_v[pl.ds(start + g * 16, 16)]
            ls = plsc.load_gather(lse_v, [ii])
            acc = acc + (ls - tl)
        return acc

    acc = lax.fori_loop(0, NCH, chunk_body, jnp.zeros((16,), jnp.float32))
    part_v[...] = acc
    pltpu.sync_copy(part_v, part_hbm.at[wid])


_sc_gather = functools.partial(
    pl.kernel,
    out_type=[
        jax.ShapeDtypeStruct((ROWS_N, VOCAB_N), jnp.float32),
        jax.ShapeDtypeStruct((NW, 16), jnp.float32),
    ],
    mesh=plsc.VectorSubcoreMesh(core_axis_name="c", subcore_axis_name="s"),
    compiler_params=pltpu.CompilerParams(use_tc_tiling_on_sc=False,
                                         needs_layout_passes=False),
    scratch_types=[
        pltpu.VMEM((PER_W,), jnp.int32),
        pltpu.VMEM((PER_W,), jnp.int32),
        pltpu.VMEM((LSE_PAD,), jnp.float32),
        pltpu.VMEM((CH, VOCAB_N), jnp.float32),
        pltpu.VMEM((16,), jnp.float32),
        pltpu.SemaphoreType.DMA,
    ],
)(_sc_body)


def kernel(idx, targets, table):
    idx_f = idx.reshape(-1).astype(jnp.int32)
    tgt_f = targets.reshape(-1).astype(jnp.int32)
    tpad = jnp.pad(table, ((0, LSE_PAD - VOCAB_N), (0, LSE_PAD - VOCAB_N)),
                   constant_values=-1e30)
    lse = pl.pallas_call(
        _lse_body,
        out_shape=jax.ShapeDtypeStruct((LSE_PAD,), jnp.float32),
    )(tpad)
    logits, part = _sc_gather(idx_f, tgt_f, lse, table)
    loss = pl.pallas_call(
        _mean_body,
        out_shape=jax.ShapeDtypeStruct((1, 1), jnp.float32),
    )(part)
    return logits, loss[0, 0]


# speedup vs baseline: 1.6984x; 1.0572x over previous
import functools

import jax
import jax.numpy as jnp
from jax import lax
from jax.experimental import pallas as pl
from jax.experimental.pallas import tpu as pltpu
from jax.experimental.pallas import tpu_sc as plsc

VOCAB_N = 1000
ROWS_N = 51200
NC = 2
NS = 16
NW = NC * NS
PER_W = ROWS_N // NW
CH = 32
NCH = PER_W // CH
LSE_PAD = 1024


def _lse_body(tpad_ref, out_ref):
    x = tpad_ref[...]
    m = jnp.max(x, axis=1)
    s = jnp.sum(jnp.exp(x - m[:, None]), axis=1)
    out_ref[...] = m + jnp.log(s)


def _mean_body(part_ref, out_ref):
    out_ref[...] = (jnp.sum(part_ref[...]) * (1.0 / ROWS_N))[None, None]


def _sc_body(idx_hbm, tgt_hbm, lse_hbm, table_hbm, out_hbm, part_hbm,
             idx_v, tgt_v, lse_v, rows_v, part_v, sem):
    wid = lax.axis_index("s") * NC + lax.axis_index("c")
    base = wid * PER_W
    pltpu.sync_copy(idx_hbm.at[pl.ds(base, PER_W)], idx_v)
    pltpu.sync_copy(tgt_hbm.at[pl.ds(base, PER_W)], tgt_v)
    pltpu.sync_copy(lse_hbm, lse_v)

    def chunk_body(c, acc):
        start = c * CH
        pltpu.async_copy(
            table_hbm.at[idx_v.at[pl.ds(start, CH)]], rows_v, sem).wait()
        pltpu.sync_copy(rows_v, out_hbm.at[pl.ds(base + start, CH)])
        for g in range(CH // 16):
            rid = lax.iota(jnp.int32, 16) + g * 16
            cid = tgt_v[pl.ds(start + g * 16, 16)]
            tl = plsc.load_gather(rows_v, [rid, cid])
            ii = idx_v[pl.ds(start + g * 16, 16)]
            ls = plsc.load_gather(lse_v, [ii])
            acc = acc + (ls - tl)
        return acc

    acc = lax.fori_loop(0, NCH, chunk_body, jnp.zeros((16,), jnp.float32))
    part_v[...] = acc
    pltpu.sync_copy(part_v, part_hbm.at[wid])


def _sc_body2(idx_hbm, tgt_hbm, lse_hbm, table_hbm, out_hbm, part_hbm,
              idx_v, tgt_v, lse_v, rows0, rows1, part_v,
              semg0, semg1, semw0, semw1):
    wid = lax.axis_index("s") * NC + lax.axis_index("c")
    base = wid * PER_W
    pltpu.sync_copy(idx_hbm.at[pl.ds(base, PER_W)], idx_v)
    pltpu.sync_copy(tgt_hbm.at[pl.ds(base, PER_W)], tgt_v)
    pltpu.sync_copy(lse_hbm, lse_v)

    rows = (rows0, rows1)
    semg = (semg0, semg1)
    semw = (semw0, semw1)

    def gather_chunk(c, b):
        return pltpu.async_copy(
            table_hbm.at[idx_v.at[pl.ds(c * CH, CH)]], rows[b], semg[b])

    gather_chunk(0, 0)
    gather_chunk(1, 1)

    def outer(o, acc):
        for b in range(2):
            c = o * 2 + b
            pltpu.make_async_copy(
                table_hbm.at[idx_v.at[pl.ds(c * CH, CH)]],
                rows[b], semg[b]).wait()
            wr = pltpu.async_copy(
                rows[b], out_hbm.at[pl.ds(base + c * CH, CH)], semw[b])
            for g in range(CH // 16):
                rid = lax.iota(jnp.int32, 16) + g * 16
                cid = tgt_v[pl.ds(c * CH + g * 16, 16)]
                tl = plsc.load_gather(rows[b], [rid, cid])
                ii = idx_v[pl.ds(c * CH + g * 16, 16)]
                ls = plsc.load_gather(lse_v, [ii])
                acc = acc + (ls - tl)
            wr.wait()

            @pl.when(c + 2 < NCH)
            def _():
                gather_chunk(c + 2, b)

        return acc

    acc = lax.fori_loop(0, NCH // 2, outer, jnp.zeros((16,), jnp.float32))
    part_v[...] = acc
    pltpu.sync_copy(part_v, part_hbm.at[wid])


_sc_gather = functools.partial(
    pl.kernel,
    out_type=[
        jax.ShapeDtypeStruct((ROWS_N, VOCAB_N), jnp.float32),
        jax.ShapeDtypeStruct((NW, 16), jnp.float32),
    ],
    mesh=plsc.VectorSubcoreMesh(core_axis_name="c", subcore_axis_name="s"),
    compiler_params=pltpu.CompilerParams(use_tc_tiling_on_sc=False,
                                         needs_layout_passes=False),
    scratch_types=[
        pltpu.VMEM((PER_W,), jnp.int32),
        pltpu.VMEM((PER_W,), jnp.int32),
        pltpu.VMEM((LSE_PAD,), jnp.float32),
        pltpu.VMEM((CH, VOCAB_N), jnp.float32),
        pltpu.VMEM((CH, VOCAB_N), jnp.float32),
        pltpu.VMEM((16,), jnp.float32),
        pltpu.SemaphoreType.DMA,
        pltpu.SemaphoreType.DMA,
        pltpu.SemaphoreType.DMA,
        pltpu.SemaphoreType.DMA,
    ],
)(_sc_body2)


def kernel(idx, targets, table):
    idx_f = idx.reshape(-1).astype(jnp.int32)
    tgt_f = targets.reshape(-1).astype(jnp.int32)
    tpad = jnp.pad(table, ((0, LSE_PAD - VOCAB_N), (0, LSE_PAD - VOCAB_N)),
                   constant_values=-1e30)
    lse = pl.pallas_call(
        _lse_body,
        out_shape=jax.ShapeDtypeStruct((LSE_PAD,), jnp.float32),
    )(tpad)
    logits, part = _sc_gather(idx_f, tgt_f, lse, table)
    loss = pl.pallas_call(
        _mean_body,
        out_shape=jax.ShapeDtypeStruct((1, 1), jnp.float32),
    )(part)
    return logits, loss[0, 0]


# speedup vs baseline: 2.2940x; 1.3506x over previous
import functools

import jax
import jax.numpy as jnp
from jax import lax
from jax.experimental import pallas as pl
from jax.experimental.pallas import tpu as pltpu
from jax.experimental.pallas import tpu_sc as plsc

VOCAB_N = 1000
VPAD = 1024
ROWS_N = 51200
NC = 2
NS = 16
NW = NC * NS
PER_W = ROWS_N // NW
CH = 32
NCH = PER_W // CH
LCH = 80


def _lse_body(tpad_ref, out_ref):
    x = tpad_ref[...]
    m = jnp.max(x, axis=1)
    s = jnp.sum(jnp.exp(x - m[:, None]), axis=1)
    out_ref[...] = m + jnp.log(s)


def _mean_body(part_ref, out_ref):
    out_ref[...] = (jnp.sum(part_ref[...]) * (1.0 / ROWS_N))[None, None]


def _sc_body(idx_hbm, tgt_hbm, lse_hbm, tflat_hbm, tpad_hbm,
             out_hbm, part_hbm,
             idx_v, tgt_v, fidx_v, lsei_v, tlog_v, rows0, rows1, part_v,
             semg0, semg1, semw0, semw1, seml):
    wid = lax.axis_index("s") * NC + lax.axis_index("c")
    base = wid * PER_W
    pltpu.sync_copy(idx_hbm.at[pl.ds(base, PER_W)], idx_v)
    pltpu.sync_copy(tgt_hbm.at[pl.ds(base, PER_W)], tgt_v)

    rows = (rows0, rows1)
    semg = (semg0, semg1)
    semw = (semw0, semw1)

    def gather_chunk(c, b):
        return pltpu.async_copy(
            tpad_hbm.at[idx_v.at[pl.ds(c * CH, CH)]], rows[b], semg[b])

    gather_chunk(0, 0)
    gather_chunk(1, 1)

    def fidx_body(j, _):
        s = pl.ds(j * 16, 16)
        fidx_v[s] = idx_v[s] * VPAD + tgt_v[s]
        return 0

    lax.fori_loop(0, PER_W // 16, fidx_body, 0)

    def lgather_body(j, _):
        s = pl.ds(j * LCH, LCH)
        pltpu.async_copy(lse_hbm.at[idx_v.at[s]], lsei_v.at[s], seml).wait()
        pltpu.async_copy(tflat_hbm.at[fidx_v.at[s]], tlog_v.at[s],
                         seml).wait()
        return 0

    lax.fori_loop(0, PER_W // LCH, lgather_body, 0)

    def loss_body(j, acc):
        s = pl.ds(j * 16, 16)
        return acc + (lsei_v[s] - tlog_v[s])

    acc = lax.fori_loop(0, PER_W // 16, loss_body,
                        jnp.zeros((16,), jnp.float32))
    part_v[...] = acc
    pltpu.sync_copy(part_v, part_hbm.at[wid])

    tail = pl.multiple_of(wid * 0 + 7 * 128, 128)

    def outer(o, _):
        for b in range(2):
            c = o * 2 + b
            pltpu.make_async_copy(
                tpad_hbm.at[idx_v.at[pl.ds(c * CH, CH)]],
                rows[b], semg[b]).wait()
            wrs = []
            for j in range(7):
                wrs.append(pltpu.async_copy(
                    rows[b].at[:, pl.ds(j * 128, 128)],
                    out_hbm.at[pl.ds(base + c * CH, CH),
                               pl.ds(j * 128, 128)], semw[b]))
            wrs.append(pltpu.async_copy(
                rows[b].at[:, pl.ds(7 * 128, 128)],
                out_hbm.at[pl.ds(base + c * CH, CH),
                           pl.ds(tail, 128)], semw[b]))
            for wr in wrs:
                wr.wait()

            @pl.when(c + 2 < NCH)
            def _():
                gather_chunk(c + 2, b)

        return 0

    lax.fori_loop(0, NCH // 2, outer, 0)


_sc_gather = functools.partial(
    pl.kernel,
    out_type=[
        jax.ShapeDtypeStruct((ROWS_N, VOCAB_N), jnp.float32),
        jax.ShapeDtypeStruct((NW, 16), jnp.float32),
    ],
    mesh=plsc.VectorSubcoreMesh(core_axis_name="c", subcore_axis_name="s"),
    compiler_params=pltpu.CompilerParams(use_tc_tiling_on_sc=True,
                                         needs_layout_passes=False,
                                         disable_bounds_checks=True),
    scratch_types=[
        pltpu.VMEM((PER_W,), jnp.int32),
        pltpu.VMEM((PER_W,), jnp.int32),
        pltpu.VMEM((PER_W,), jnp.int32),
        pltpu.VMEM((PER_W,), jnp.float32),
        pltpu.VMEM((PER_W,), jnp.float32),
        pltpu.VMEM((CH, VPAD), jnp.float32),
        pltpu.VMEM((CH, VPAD), jnp.float32),
        pltpu.VMEM((16,), jnp.float32),
        pltpu.SemaphoreType.DMA,
        pltpu.SemaphoreType.DMA,
        pltpu.SemaphoreType.DMA,
        pltpu.SemaphoreType.DMA,
        pltpu.SemaphoreType.DMA,
    ],
)(_sc_body)


def kernel(idx, targets, table):
    idx_f = idx.reshape(-1).astype(jnp.int32)
    tgt_f = targets.reshape(-1).astype(jnp.int32)
    cpad = VPAD - VOCAB_N
    tpad_sq = jnp.pad(table, ((0, cpad), (0, cpad)), constant_values=-1e30)
    tpad_g = jnp.pad(table, ((0, 0), (0, cpad)), constant_values=-2e30)
    tflat = jnp.pad(table, ((0, 0), (0, cpad))).reshape(-1)
    lse = pl.pallas_call(
        _lse_body,
        out_shape=jax.ShapeDtypeStruct((VPAD,), jnp.float32),
    )(tpad_sq)
    logits, part = _sc_gather(idx_f, tgt_f, lse, tflat, tpad_g)
    loss = pl.pallas_call(
        _mean_body,
        out_shape=jax.ShapeDtypeStruct((1, 1), jnp.float32),
    )(part)
    return logits, loss[0, 0]
